# baseline (speedup 1.0000x reference)
import jax
import jax.numpy as jnp
from jax.experimental import pallas as pl
from jax.experimental.pallas import tpu as pltpu

B, M, H, D, K = 8, 512, 8, 32, 16
TOT = B * M * H * D * D
ROWS = TOT // 128
NROWS = B * M * H * D // 128
CHR = 32768
NCH = ROWS // CHR
NBUF = 3


def _body(sel_ref, probs_ref, mat_in, norm_in, mat_out, norm_out,
          buf, nbuf, in_sems, out_sems, nsem):
    ncp = pltpu.make_async_copy(norm_in, nbuf, nsem)
    ncp.start()

    def start_in(c):
        pltpu.make_async_copy(mat_in.at[pl.ds(c * CHR, CHR)],
                              buf.at[c % NBUF], in_sems.at[c % NBUF]).start()

    def wait_in(c):
        pltpu.make_async_copy(mat_in.at[pl.ds(c * CHR, CHR)],
                              buf.at[c % NBUF], in_sems.at[c % NBUF]).wait()

    def start_out(c):
        pltpu.make_async_copy(buf.at[c % NBUF], mat_out.at[pl.ds(c * CHR, CHR)],
                              out_sems.at[c % NBUF]).start()

    def wait_out(c):
        pltpu.make_async_copy(buf.at[c % NBUF], mat_out.at[pl.ds(c * CHR, CHR)],
                              out_sems.at[c % NBUF]).wait()

    for c in range(NBUF):
        start_in(c)
    for c in range(NCH):
        wait_in(c)
        start_out(c)
        wait_out(c)
        if c + NBUF < NCH:
            start_in(c + NBUF)

    ncp.wait()
    pltpu.make_async_copy(nbuf, norm_out, nsem).start()
    pltpu.make_async_copy(nbuf, norm_out, nsem).wait()


def kernel(matrix, normalizer, matrix_update, normalizer_update,
           main_decay_logits, aux_decay_logits, sel_index, sel_probs):
    mat2 = matrix.reshape(ROWS, 128)
    norm2 = normalizer.reshape(NROWS, 128)

    grid_spec = pltpu.PrefetchScalarGridSpec(
        num_scalar_prefetch=2,
        grid=(1,),
        in_specs=[
            pl.BlockSpec(memory_space=pl.ANY),
            pl.BlockSpec(memory_space=pl.ANY),
        ],
        out_specs=[
            pl.BlockSpec(memory_space=pl.ANY),
            pl.BlockSpec(memory_space=pl.ANY),
        ],
        scratch_shapes=[
            pltpu.VMEM((NBUF, CHR, 128), jnp.float32),
            pltpu.VMEM((NROWS, 128), jnp.float32),
            pltpu.SemaphoreType.DMA((NBUF,)),
            pltpu.SemaphoreType.DMA((NBUF,)),
            pltpu.SemaphoreType.DMA,
        ],
    )

    out_mat, out_norm = pl.pallas_call(
        _body,
        grid_spec=grid_spec,
        out_shape=[
            jax.ShapeDtypeStruct(mat2.shape, mat2.dtype),
            jax.ShapeDtypeStruct(norm2.shape, norm2.dtype),
        ],
    )(sel_index, sel_probs, mat2, norm2)

    return (out_mat.reshape(B, M, H, D, D), out_norm.reshape(B, M, H, D))


# speedup vs baseline: 1.3893x; 1.3893x over previous
import jax
import jax.numpy as jnp
from jax.experimental import pallas as pl
from jax.experimental.pallas import tpu as pltpu

B, M, H, D, K = 8, 512, 8, 32, 16
MB = 64
NB = M // MB


def _body(m_s, h_s, k_s, starts, ends, probs,
          mat_in, norm_in, mu_ref, nu_ref, main_ref, aux_ref,
          mat_out, norm_out):
    b = pl.program_id(0)
    nb = pl.program_id(1)

    mat_out[...] = mat_in[...]
    norm_out[...] = norm_in[...]

    def upd(i, carry):
        m = m_s[b, i]
        h = h_s[b, i]
        k = k_s[b, i]
        m_rel = m - nb * MB
        p = probs[b, h, k]

        mrow = main_ref[pl.ds(m, 1), h]
        mcol = jnp.swapaxes(mrow, 0, 1)
        arow = aux_ref[pl.ds(m, 1)]
        mat_dec = jax.nn.sigmoid(mcol + arow)
        norm_dec = jax.nn.sigmoid(mrow)

        sel_m = mat_out[0, m_rel, h]
        mu = mu_ref[0, k, h]
        mat_out[0, m_rel, h] = sel_m + (p * mat_dec) * (mu - sel_m)

        sel_n = norm_out[0, pl.ds(m_rel, 1), h]
        nu = nu_ref[0, pl.ds(k, 1), h]
        norm_out[0, pl.ds(m_rel, 1), h] = sel_n + (p * norm_dec) * (nu - sel_n)
        return carry

    jax.lax.fori_loop(starts[b, nb], ends[b, nb], upd, 0)


def kernel(matrix, normalizer, matrix_update, normalizer_update,
           main_decay_logits, aux_decay_logits, sel_index, sel_probs):
    aux2 = aux_decay_logits.reshape(M, D)

    m_all = sel_index.reshape(B, H * K)
    order = jnp.argsort(m_all, axis=1).astype(jnp.int32)
    m_sorted = jnp.take_along_axis(m_all, order, axis=1).astype(jnp.int32)
    h_sorted = order // K
    k_sorted = order % K
    bounds = jnp.arange(NB + 1, dtype=jnp.int32) * MB
    pos = jax.vmap(lambda row: jnp.searchsorted(row, bounds, side='left'))(
        m_sorted).astype(jnp.int32)
    starts, ends = pos[:, :-1], pos[:, 1:]

    def mem_map(b, nb, *_):
        return (b, nb, 0, 0, 0)

    def nrm_map(b, nb, *_):
        return (b, nb, 0, 0)

    def upd_map(b, nb, *_):
        return (b, 0, 0, 0, 0)

    def upd_nrm_map(b, nb, *_):
        return (b, 0, 0, 0)

    def whole3(*_):
        return (0, 0, 0)

    def whole2(*_):
        return (0, 0)

    grid_spec = pltpu.PrefetchScalarGridSpec(
        num_scalar_prefetch=6,
        grid=(B, NB),
        in_specs=[
            pl.BlockSpec((1, MB, H, D, D), mem_map),
            pl.BlockSpec((1, MB, H, D), nrm_map),
            pl.BlockSpec((1, K, H, D, D), upd_map),
            pl.BlockSpec((1, K, H, D), upd_nrm_map),
            pl.BlockSpec((M, H, D), whole3),
            pl.BlockSpec((M, D), whole2),
        ],
        out_specs=[
            pl.BlockSpec((1, MB, H, D, D), mem_map),
            pl.BlockSpec((1, MB, H, D), nrm_map),
        ],
    )

    out_mat, out_norm = pl.pallas_call(
        _body,
        grid_spec=grid_spec,
        out_shape=[
            jax.ShapeDtypeStruct(matrix.shape, matrix.dtype),
            jax.ShapeDtypeStruct(normalizer.shape, normalizer.dtype),
        ],
    )(m_sorted, h_sorted, k_sorted, starts, ends, sel_probs,
      matrix, normalizer, matrix_update, normalizer_update,
      main_decay_logits, aux2)

    return (out_mat, out_norm)


# speedup vs baseline: 1.5504x; 1.1159x over previous
import jax
import jax.numpy as jnp
from jax.experimental import pallas as pl
from jax.experimental.pallas import tpu as pltpu

B, M, H, D, K = 8, 512, 8, 32, 16
NSLOT = B * H * K
NCHUNK = 2
MC = M // NCHUNK


def _body(sel_ref, probs_ref,
          mat_in, norm_in, mu_in, nu_ref, main_ref, aux_ref,
          mat_out, norm_out,
          slot_buf, mu_buf, norm_buf,
          nload_sem, gather_sem, scatter_sem, nstore_sem):

    pltpu.make_async_copy(norm_in, norm_buf, nload_sem).start()

    def issue_gathers(i, carry):
        r = i % (H * K)
        b = i // (H * K)
        h = r // K
        k = r % K
        m = sel_ref[b, h, k]
        pltpu.make_async_copy(mat_in.at[b, m, h], slot_buf.at[i],
                              gather_sem).start()
        pltpu.make_async_copy(mu_in.at[b, k, h], mu_buf.at[i],
                              gather_sem).start()
        return carry
    jax.lax.fori_loop(0, NSLOT, issue_gathers, 0)

    pltpu.make_async_copy(norm_in, norm_buf, nload_sem).wait()

    def norm_upd(i, carry):
        r = i % (H * K)
        b = i // (H * K)
        h = r // K
        k = r % K
        m = sel_ref[b, h, k]
        p = probs_ref[b, h, k]
        mrow = main_ref[pl.ds(m, 1), h]
        dec = jax.nn.sigmoid(mrow)
        sel_n = norm_buf[b, pl.ds(m, 1), h]
        nu = nu_ref[b, pl.ds(k, 1), h]
        norm_buf[b, pl.ds(m, 1), h] = sel_n + (p * dec) * (nu - sel_n)
        return carry
    jax.lax.fori_loop(0, NSLOT, norm_upd, 0)

    pltpu.make_async_copy(norm_buf, norm_out, nstore_sem).start()

    def wait_gathers(i, carry):
        pltpu.make_async_copy(mat_in.at[0, 0, 0], slot_buf.at[0],
                              gather_sem).wait()
        pltpu.make_async_copy(mu_in.at[0, 0, 0], mu_buf.at[0],
                              gather_sem).wait()
        return carry
    jax.lax.fori_loop(0, NSLOT, wait_gathers, 0)

    def blend(i, carry):
        r = i % (H * K)
        b = i // (H * K)
        h = r // K
        k = r % K
        m = sel_ref[b, h, k]
        p = probs_ref[b, h, k]
        mrow = main_ref[pl.ds(m, 1), h]
        mcol = jnp.swapaxes(mrow, 0, 1)
        arow = aux_ref[pl.ds(m, 1)]
        dec = jax.nn.sigmoid(mcol + arow)
        sel_m = slot_buf[i]
        mu = mu_buf[i]
        slot_buf[i] = sel_m + (p * dec) * (mu - sel_m)
        return carry
    jax.lax.fori_loop(0, NSLOT, blend, 0)

    def scatter(i, carry):
        r = i % (H * K)
        b = i // (H * K)
        h = r // K
        k = r % K
        m = sel_ref[b, h, k]
        pltpu.make_async_copy(slot_buf.at[i], mat_out.at[b, m, h],
                              scatter_sem).start()
        return carry
    jax.lax.fori_loop(0, NSLOT, scatter, 0)

    def wait_scatter(i, carry):
        pltpu.make_async_copy(slot_buf.at[0], mat_out.at[0, 0, 0],
                              scatter_sem).wait()
        return carry
    jax.lax.fori_loop(0, NSLOT, wait_scatter, 0)

    pltpu.make_async_copy(norm_buf, norm_out, nstore_sem).wait()


def kernel(matrix, normalizer, matrix_update, normalizer_update,
           main_decay_logits, aux_decay_logits, sel_index, sel_probs):
    aux2 = aux_decay_logits.reshape(M, D)

    def whole(*_):
        return tuple(0 for _ in range(4))

    grid_spec = pltpu.PrefetchScalarGridSpec(
        num_scalar_prefetch=2,
        grid=(1,),
        in_specs=[
            pl.BlockSpec(memory_space=pl.ANY),
            pl.BlockSpec(memory_space=pl.ANY),
            pl.BlockSpec(memory_space=pl.ANY),
            pl.BlockSpec((B, K, H, D), lambda i, *_: (0, 0, 0, 0)),
            pl.BlockSpec((M, H, D), lambda i, *_: (0, 0, 0)),
            pl.BlockSpec((M, D), lambda i, *_: (0, 0)),
        ],
        out_specs=[
            pl.BlockSpec(memory_space=pl.ANY),
            pl.BlockSpec(memory_space=pl.ANY),
        ],
        scratch_shapes=[
            pltpu.VMEM((NSLOT, D, D), jnp.float32),
            pltpu.VMEM((NSLOT, D, D), jnp.float32),
            pltpu.VMEM((B, M, H, D), jnp.float32),
            pltpu.SemaphoreType.DMA,
            pltpu.SemaphoreType.DMA,
            pltpu.SemaphoreType.DMA,
            pltpu.SemaphoreType.DMA,
        ],
    )

    out_mat, out_norm = pl.pallas_call(
        _body,
        grid_spec=grid_spec,
        out_shape=[
            jax.ShapeDtypeStruct(matrix.shape, matrix.dtype),
            jax.ShapeDtypeStruct(normalizer.shape, normalizer.dtype),
        ],
        input_output_aliases={2: 0},
    )(sel_index, sel_probs,
      matrix, normalizer, matrix_update, normalizer_update,
      main_decay_logits, aux2)

    return (out_mat, out_norm)


# speedup vs baseline: 1.5901x; 1.0256x over previous
import jax
import jax.numpy as jnp
from jax.experimental import pallas as pl
from jax.experimental.pallas import tpu as pltpu

B, M, H, D, K = 8, 512, 8, 32, 16
NSLOT = B * H * K
NCHUNK = 2
MC = M // NCHUNK


def _body(sel_ref, probs_ref,
          mat_in, norm_in, mu_in, nu_ref, main_ref, aux_ref,
          mat_out, norm_out,
          slot_buf, mu_buf, norm_buf,
          nload_sem, gather_sem, scatter_sem, nstore_sem):

    pltpu.make_async_copy(norm_in, norm_buf, nload_sem).start()

    def issue_gathers(i, carry):
        r = i % (H * K)
        b = i // (H * K)
        h = r // K
        k = r % K
        m = sel_ref[b, h, k]
        pltpu.make_async_copy(mat_in.at[b, m, h], slot_buf.at[i],
                              gather_sem).start()
        pltpu.make_async_copy(mu_in.at[b, k, h], mu_buf.at[i],
                              gather_sem).start()
        return carry
    jax.lax.fori_loop(0, NSLOT, issue_gathers, 0)

    pltpu.make_async_copy(norm_in, norm_buf, nload_sem).wait()

    def wait_gathers(i, carry):
        pltpu.make_async_copy(mat_in.at[0, 0, 0], slot_buf.at[0],
                              gather_sem).wait()
        pltpu.make_async_copy(mu_in.at[0, 0, 0], mu_buf.at[0],
                              gather_sem).wait()
        return carry
    jax.lax.fori_loop(0, NSLOT, wait_gathers, 0)

    def blend(i, carry):
        r = i % (H * K)
        b = i // (H * K)
        h = r // K
        k = r % K
        m = sel_ref[b, h, k]
        p = probs_ref[b, h, k]
        mrow = main_ref[pl.ds(m, 1), h]
        mcol = jnp.swapaxes(mrow, 0, 1)
        arow = aux_ref[pl.ds(m, 1)]
        dec = jax.nn.sigmoid(mcol + arow)
        sel_m = slot_buf[i]
        mu = mu_buf[i]
        slot_buf[i] = sel_m + (p * dec) * (mu - sel_m)
        ndec = jax.nn.sigmoid(mrow)
        sel_n = norm_buf[b, pl.ds(m, 1), h]
        nu = nu_ref[b, pl.ds(k, 1), h]
        norm_buf[b, pl.ds(m, 1), h] = sel_n + (p * ndec) * (nu - sel_n)
        return carry
    jax.lax.fori_loop(0, NSLOT, blend, 0)

    pltpu.make_async_copy(norm_buf, norm_out, nstore_sem).start()

    def scatter(i, carry):
        r = i % (H * K)
        b = i // (H * K)
        h = r // K
        k = r % K
        m = sel_ref[b, h, k]
        pltpu.make_async_copy(slot_buf.at[i], mat_out.at[b, m, h],
                              scatter_sem).start()
        return carry
    jax.lax.fori_loop(0, NSLOT, scatter, 0)

    def wait_scatter(i, carry):
        pltpu.make_async_copy(slot_buf.at[0], mat_out.at[0, 0, 0],
                              scatter_sem).wait()
        return carry
    jax.lax.fori_loop(0, NSLOT, wait_scatter, 0)

    pltpu.make_async_copy(norm_buf, norm_out, nstore_sem).wait()


def kernel(matrix, normalizer, matrix_update, normalizer_update,
           main_decay_logits, aux_decay_logits, sel_index, sel_probs):
    aux2 = aux_decay_logits.reshape(M, D)

    def whole(*_):
        return tuple(0 for _ in range(4))

    grid_spec = pltpu.PrefetchScalarGridSpec(
        num_scalar_prefetch=2,
        grid=(1,),
        in_specs=[
            pl.BlockSpec(memory_space=pl.ANY),
            pl.BlockSpec(memory_space=pl.ANY),
            pl.BlockSpec(memory_space=pl.ANY),
            pl.BlockSpec((B, K, H, D), lambda i, *_: (0, 0, 0, 0)),
            pl.BlockSpec((M, H, D), lambda i, *_: (0, 0, 0)),
            pl.BlockSpec((M, D), lambda i, *_: (0, 0)),
        ],
        out_specs=[
            pl.BlockSpec(memory_space=pl.ANY),
            pl.BlockSpec(memory_space=pl.ANY),
        ],
        scratch_shapes=[
            pltpu.VMEM((NSLOT, D, D), jnp.float32),
            pltpu.VMEM((NSLOT, D, D), jnp.float32),
            pltpu.VMEM((B, M, H, D), jnp.float32),
            pltpu.SemaphoreType.DMA,
            pltpu.SemaphoreType.DMA,
            pltpu.SemaphoreType.DMA,
            pltpu.SemaphoreType.DMA,
        ],
    )

    out_mat, out_norm = pl.pallas_call(
        _body,
        grid_spec=grid_spec,
        out_shape=[
            jax.ShapeDtypeStruct(matrix.shape, matrix.dtype),
            jax.ShapeDtypeStruct(normalizer.shape, normalizer.dtype),
        ],
        input_output_aliases={2: 0},
    )(sel_index, sel_probs,
      matrix, normalizer, matrix_update, normalizer_update,
      main_decay_logits, aux2)

    return (out_mat, out_norm)


# speedup vs baseline: 1.5913x; 1.0008x over previous
import jax
import jax.numpy as jnp
from jax.experimental import pallas as pl
from jax.experimental.pallas import tpu as pltpu

B, M, H, D, K = 8, 512, 8, 32, 16
NSLOT = B * H * K


def _body(sel_ref, probs_ref,
          mat_in, norm_in, mu_in, nu_ref, main_ref, aux_ref,
          mat_out, norm_out,
          slot_buf, mu_buf, norm_buf,
          nload_sem, gather_sem, scatter_sem, nstore_sem):

    pltpu.make_async_copy(norm_in, norm_buf, nload_sem).start()

    def issue_gathers(i, carry):
        r = i % (H * K)
        b = i // (H * K)
        h = r // K
        k = r % K
        m = sel_ref[b, h, k]
        pltpu.make_async_copy(mat_in.at[b, m, h], slot_buf.at[i],
                              gather_sem).start()
        pltpu.make_async_copy(mu_in.at[b, k, h], mu_buf.at[i],
                              gather_sem).start()
        return carry
    jax.lax.fori_loop(0, NSLOT, issue_gathers, 0)

    pltpu.make_async_copy(norm_in, norm_buf, nload_sem).wait()

    def wait_gathers(i, carry):
        pltpu.make_async_copy(mat_in.at[0, 0, 0], slot_buf.at[0],
                              gather_sem).wait()
        pltpu.make_async_copy(mu_in.at[0, 0, 0], mu_buf.at[0],
                              gather_sem).wait()
        return carry
    jax.lax.fori_loop(0, NSLOT, wait_gathers, 0)

    def blend(i, carry):
        r = i % (H * K)
        b = i // (H * K)
        h = r // K
        k = r % K
        m = sel_ref[b, h, k]
        p = probs_ref[b, h, k]
        mrow = main_ref[pl.ds(m, 1), h]
        mcol = jnp.swapaxes(mrow, 0, 1)
        arow = aux_ref[pl.ds(m, 1)]
        dec = jax.nn.sigmoid(mcol + arow)
        sel_m = slot_buf[i]
        mu = mu_buf[i]
        slot_buf[i] = sel_m + (p * dec) * (mu - sel_m)
        ndec = jax.nn.sigmoid(mrow)
        sel_n = norm_buf[b, pl.ds(m, 1), h]
        nu = nu_ref[b, pl.ds(k, 1), h]
        norm_buf[b, pl.ds(m, 1), h] = sel_n + (p * ndec) * (nu - sel_n)
        return carry
    jax.lax.fori_loop(0, NSLOT, blend, 0)

    pltpu.make_async_copy(norm_buf, norm_out, nstore_sem).start()

    def scatter(i, carry):
        r = i % (H * K)
        b = i // (H * K)
        h = r // K
        k = r % K
        m = sel_ref[b, h, k]
        pltpu.make_async_copy(slot_buf.at[i], mat_out.at[b, m, h],
                              scatter_sem).start()
        return carry
    jax.lax.fori_loop(0, NSLOT, scatter, 0)

    def wait_scatter(i, carry):
        pltpu.make_async_copy(slot_buf.at[0], mat_out.at[0, 0, 0],
                              scatter_sem).wait()
        return carry
    jax.lax.fori_loop(0, NSLOT, wait_scatter, 0)

    pltpu.make_async_copy(norm_buf, norm_out, nstore_sem).wait()


def kernel(matrix, normalizer, matrix_update, normalizer_update,
           main_decay_logits, aux_decay_logits, sel_index, sel_probs):
    aux2 = aux_decay_logits.reshape(M, D)

    def whole(*_):
        return tuple(0 for _ in range(4))

    grid_spec = pltpu.PrefetchScalarGridSpec(
        num_scalar_prefetch=2,
        grid=(1,),
        in_specs=[
            pl.BlockSpec(memory_space=pl.ANY),
            pl.BlockSpec(memory_space=pl.ANY),
            pl.BlockSpec(memory_space=pl.ANY),
            pl.BlockSpec((B, K, H, D), lambda i, *_: (0, 0, 0, 0)),
            pl.BlockSpec((M, H, D), lambda i, *_: (0, 0, 0)),
            pl.BlockSpec((M, D), lambda i, *_: (0, 0)),
        ],
        out_specs=[
            pl.BlockSpec(memory_space=pl.ANY),
            pl.BlockSpec(memory_space=pl.ANY),
        ],
        scratch_shapes=[
            pltpu.VMEM((NSLOT, D, D), jnp.float32),
            pltpu.VMEM((NSLOT, D, D), jnp.float32),
            pltpu.VMEM((B, M, H, D), jnp.float32),
            pltpu.SemaphoreType.DMA,
            pltpu.SemaphoreType.DMA,
            pltpu.SemaphoreType.DMA,
            pltpu.SemaphoreType.DMA,
        ],
    )

    out_mat, out_norm = pl.pallas_call(
        _body,
        grid_spec=grid_spec,
        out_shape=[
            jax.ShapeDtypeStruct(matrix.shape, matrix.dtype),
            jax.ShapeDtypeStruct(normalizer.shape, normalizer.dtype),
        ],
        input_output_aliases={2: 0},
    )(sel_index, sel_probs,
      matrix, normalizer, matrix_update, normalizer_update,
      main_decay_logits, aux2)

    return (out_mat, out_norm)
